# Initial kernel scaffold; baseline (speedup 1.0000x reference)
import jax
import jax.numpy as jnp
from jax.experimental import pallas as pl


def kernel(x, edge_index, W_F1, asrc_F1, adst_F1, b_F1, W_F2, asrc_F2, adst_F2, b_F2, W_G1, asrc_G1, adst_G1, b_G1, W_G2, asrc_G2, adst_G2, b_G2):
    raise NotImplementedError("write your pallas kernel here")


# speedup vs baseline: 33.4829x; 33.4829x over previous
import functools

import jax
import jax.numpy as jnp
from jax import lax
from jax.experimental import pallas as pl
from jax.experimental.pallas import tpu as pltpu
from jax.experimental.pallas import tpu_sc as plsc

D = 16
NSC = 2
NTILE = 16
CH = 128
ROWPT = 3200
ZR = 400


def _prep_body(x_ref, wcat_ref, adcat_ref, h0, h1, h2, h3, ad_ref):
    xb = x_ref[...]
    x1 = xb[:, :D]
    x2 = xb[:, D:]
    outs = (h0, h1, h2, h3)
    cols = []
    for c in range(4):
        xh = x1 if c < 2 else x2
        h = jnp.dot(xh, wcat_ref[c], preferred_element_type=jnp.float32)
        outs[c][...] = h
        cols.append((h * adcat_ref[c][None, :]).sum(-1, keepdims=True))
    zero = jnp.zeros((xb.shape[0], D - 4), jnp.float32)
    ad_ref[...] = jnp.concatenate(cols + [zero], axis=1)


def _fin_body(acc_ref, x_ref, btab_ref, x1n_ref, x2n_ref, ld_ref):
    def conv_out(c):
        a = acc_ref[c]
        return a[:, :D] / a[:, D:D + 1] + btab_ref[c][None, :]

    s1 = conv_out(0)
    t1 = conv_out(1)
    s2 = conv_out(2)
    t2 = conv_out(3)
    x2 = x_ref[:, D:]
    x1n = x2 * jnp.exp(s1) + t1
    x2n = x1n * jnp.exp(s2) + t2
    x1n_ref[...] = x1n
    x2n_ref[...] = x2n
    ld_ref[...] = (s1 + s2).sum(axis=1, keepdims=True)


def _sc_body(hcat, adtab, asrct, srcs, dsts, zeros_h, out,
             acc, si, di, hrows, adrows, sbuf, asrcv, semh, semad,
             chunks_per_tile, npad):
    cid = lax.axis_index("c")
    sid = lax.axis_index("s")
    iota = lax.iota(jnp.int32, D)

    pltpu.sync_copy(zeros_h.at[pl.ds(0, CH)], sbuf)

    for p in range(2):
        conv = 2 * cid + p

        def zero_body(z, _):
            r0 = sid * ROWPT + z * ZR
            pltpu.sync_copy(zeros_h, acc.at[pl.ds(r0, ZR)])
            return 0
        lax.fori_loop(0, ROWPT // ZR, zero_body, 0)
        plsc.subcore_barrier()

        pltpu.sync_copy(asrct.at[conv], asrcv)
        av = asrcv[...]
        asrc_s = [av[c] for c in range(D)]
        hoff = jnp.broadcast_to(conv * npad, (D,)).astype(jnp.int32)

        def chunk_body(i, _):
            base = (sid * chunks_per_tile + i) * CH
            pltpu.sync_copy(srcs.at[pl.ds(base, CH)], si)
            pltpu.sync_copy(dsts.at[pl.ds(base, CH)], di)
            for g in range(CH // D):
                si[pl.ds(g * D, D)] = si[pl.ds(g * D, D)] + hoff
            cph = pltpu.async_copy(hcat.at[si], hrows, semh)
            cpa = pltpu.async_copy(adtab.at[di], adrows, semad)
            cph.wait()
            cpa.wait()
            ccol = jnp.broadcast_to(conv, (D,)).astype(jnp.int32)
            for g in range(CH // D):
                ridx = iota + g * D
                a_s = jnp.zeros((D,), jnp.float32)
                hcols = []
                for c in range(D):
                    hc = plsc.load_gather(
                        hrows, [ridx, jnp.full((D,), c, jnp.int32)])
                    hcols.append(hc)
                    a_s = a_s + hc * asrc_s[c]
                a_d = plsc.load_gather(adrows, [ridx, ccol])
                al = a_s + a_d
                al = jnp.where(al >= 0, al, al * jnp.float32(0.2))
                w = jnp.exp(al)
                plsc.store_scatter(
                    sbuf, [ridx, jnp.full((D,), D, jnp.int32)], w)
                for c in range(D):
                    plsc.store_scatter(
                        sbuf, [ridx, jnp.full((D,), c, jnp.int32)],
                        hcols[c] * w)
            pltpu.sync_copy(sbuf, acc.at[di], add=True)
            return 0

        lax.fori_loop(0, chunks_per_tile, chunk_body, 0)
        plsc.subcore_barrier()

        def wb_body(z, _):
            r0 = sid * ROWPT + z * ZR
            pltpu.sync_copy(acc.at[pl.ds(r0, ZR)],
                            out.at[conv, pl.ds(r0, ZR)])
            return 0
        lax.fori_loop(0, ROWPT // ZR, wb_body, 0)
        plsc.subcore_barrier()


def kernel(x, edge_index, W_F1, asrc_F1, adst_F1, b_F1,
           W_F2, asrc_F2, adst_F2, b_F2,
           W_G1, asrc_G1, adst_G1, b_G1,
           W_G2, asrc_G2, adst_G2, b_G2):
    n = x.shape[0]
    e = edge_index.shape[1]
    npad = ((n + NTILE * ZR - 1) // (NTILE * ZR)) * (NTILE * ZR)
    if npad == n:
        npad += NTILE * ZR
    et = e + n
    chunks_per_tile = -(-et // (CH * NTILE))
    et_pad = chunks_per_tile * CH * NTILE

    idt = edge_index.dtype
    loops = jnp.arange(n, dtype=idt)
    fill = jnp.full((et_pad - et,), n, dtype=idt)
    srcs = jnp.concatenate([edge_index[0], loops, fill]).astype(jnp.int32)
    dsts = jnp.concatenate([edge_index[1], loops, fill]).astype(jnp.int32)

    x_pad = jnp.zeros((npad, 2 * D), jnp.float32).at[:n].set(x)
    wcat = jnp.stack([W_F1, W_F2, W_G1, W_G2])
    adcat = jnp.stack([adst_F1, adst_F2, adst_G1, adst_G2])
    asrct = jnp.stack([asrc_F1, asrc_F2, asrc_G1, asrc_G2])
    btab = jnp.stack([b_F1, b_F2, b_G1, b_G2])
    zeros_h = jnp.zeros((ZR, 2 * D), jnp.float32)

    blk = 256
    grid = (npad // blk,)
    h_sh = jax.ShapeDtypeStruct((npad, D), jnp.float32)
    h0, h1, h2, h3, adtab = pl.pallas_call(
        _prep_body,
        grid=grid,
        in_specs=[
            pl.BlockSpec((blk, 2 * D), lambda i: (i, 0)),
            pl.BlockSpec((4, D, D), lambda i: (0, 0, 0)),
            pl.BlockSpec((4, D), lambda i: (0, 0)),
        ],
        out_specs=[pl.BlockSpec((blk, D), lambda i: (i, 0))] * 5,
        out_shape=[h_sh, h_sh, h_sh, h_sh, h_sh],
    )(x_pad, wcat, adcat)
    hcat = jnp.concatenate([h0, h1, h2, h3], axis=0)

    mesh = plsc.VectorSubcoreMesh(core_axis_name="c", subcore_axis_name="s")
    sc = functools.partial(
        pl.kernel,
        out_type=jax.ShapeDtypeStruct((4, npad, 2 * D), jnp.float32),
        mesh=mesh,
        scratch_types=[
            pltpu.VMEM_SHARED((npad, 2 * D), jnp.float32),
            pltpu.VMEM((CH,), jnp.int32),
            pltpu.VMEM((CH,), jnp.int32),
            pltpu.VMEM((CH, D), jnp.float32),
            pltpu.VMEM((CH, D), jnp.float32),
            pltpu.VMEM((CH, 2 * D), jnp.float32),
            pltpu.VMEM((D,), jnp.float32),
            pltpu.SemaphoreType.DMA,
            pltpu.SemaphoreType.DMA,
        ],
        compiler_params=pltpu.CompilerParams(
            needs_layout_passes=False, use_tc_tiling_on_sc=False),
    )(functools.partial(_sc_body, chunks_per_tile=chunks_per_tile,
                        npad=npad))
    accs = sc(hcat, adtab, asrct, srcs, dsts, zeros_h)

    x1n, x2n, ld = pl.pallas_call(
        _fin_body,
        grid=grid,
        in_specs=[
            pl.BlockSpec((4, blk, 2 * D), lambda i: (0, i, 0)),
            pl.BlockSpec((blk, 2 * D), lambda i: (i, 0)),
            pl.BlockSpec((4, D), lambda i: (0, 0)),
        ],
        out_specs=[
            pl.BlockSpec((blk, D), lambda i: (i, 0)),
            pl.BlockSpec((blk, D), lambda i: (i, 0)),
            pl.BlockSpec((blk, 1), lambda i: (i, 0)),
        ],
        out_shape=[
            jax.ShapeDtypeStruct((npad, D), jnp.float32),
            jax.ShapeDtypeStruct((npad, D), jnp.float32),
            jax.ShapeDtypeStruct((npad, 1), jnp.float32),
        ],
    )(accs, x_pad, btab)

    return x1n[:n], x2n[:n], ld[:n, 0]


# speedup vs baseline: 52.5500x; 1.5695x over previous
import functools

import jax
import jax.numpy as jnp
from jax import lax
from jax.experimental import pallas as pl
from jax.experimental.pallas import tpu as pltpu
from jax.experimental.pallas import tpu_sc as plsc

D = 16
NSC = 2
NTILE = 16
CH = 128
ROWPT = 3200
ZR = 400


def _prep_body(x_ref, wcat_ref, adcat_ref, h0, h1, h2, h3, ad_ref):
    xb = x_ref[...]
    x1 = xb[:, :D]
    x2 = xb[:, D:]
    outs = (h0, h1, h2, h3)
    cols = []
    for c in range(4):
        xh = x1 if c < 2 else x2
        h = jnp.dot(xh, wcat_ref[c], preferred_element_type=jnp.float32)
        outs[c][...] = h
        cols.append((h * adcat_ref[c][None, :]).sum(-1, keepdims=True))
    zero = jnp.zeros((xb.shape[0], D - 4), jnp.float32)
    ad_ref[...] = jnp.concatenate(cols + [zero], axis=1)


def _fin_body(acc_ref, x_ref, btab_ref, x1n_ref, x2n_ref, ld_ref):
    def conv_out(c):
        a = acc_ref[c]
        return a[:, :D] / a[:, D:D + 1] + btab_ref[c][None, :]

    s1 = conv_out(0)
    t1 = conv_out(1)
    s2 = conv_out(2)
    t2 = conv_out(3)
    x2 = x_ref[:, D:]
    x1n = x2 * jnp.exp(s1) + t1
    x2n = x1n * jnp.exp(s2) + t2
    x1n_ref[...] = x1n
    x2n_ref[...] = x2n
    ld_ref[...] = (s1 + s2).sum(axis=1, keepdims=True)


def _sc_body(hcat, adtab, asrct, edges3, zeros_h, out,
             acc, sidi0, sidi1, gidx0, gidx1, hrows0, hrows1,
             adrows0, adrows1, sbuf0, sbuf1, asrcv,
             semi0, semi1, semh0, semh1, sema0, sema1,
             chunks_per_tile, npad):
    cid = lax.axis_index("c")
    sid = lax.axis_index("s")
    iota = lax.iota(jnp.int32, D)
    cpt = chunks_per_tile
    sets = ((sidi0, gidx0, hrows0, adrows0, sbuf0, semi0, semh0, sema0),
            (sidi1, gidx1, hrows1, adrows1, sbuf1, semi1, semh1, sema1))

    pltpu.sync_copy(zeros_h.at[pl.ds(0, CH)], sbuf0)
    pltpu.sync_copy(zeros_h.at[pl.ds(0, CH)], sbuf1)

    def fire_idx(j, s):
        sidi, _, _, _, _, semi, _, _ = s
        pltpu.async_copy(edges3.at[sid * cpt + j], sidi, semi)

    for p in range(2):
        conv = 2 * cid + p

        def zero_body(z, _):
            r0 = sid * ROWPT + z * ZR
            pltpu.sync_copy(zeros_h, acc.at[pl.ds(r0, ZR)])
            return 0
        lax.fori_loop(0, ROWPT // ZR, zero_body, 0)
        plsc.subcore_barrier()

        pltpu.sync_copy(asrct.at[conv], asrcv)
        av = asrcv[...]
        asrc_s = [av[c] for c in range(D)]
        hoff = jnp.broadcast_to(conv * npad, (D,)).astype(jnp.int32)
        ccol = jnp.broadcast_to(conv, (D,)).astype(jnp.int32)

        def prep_fire(s):
            sidi, gidx, hrows, adrows, _, semi, semh, sema = s
            pltpu.make_async_copy(edges3.at[0], sidi, semi).wait()
            for g in range(CH // D):
                d = pl.ds(g * D, D)
                gidx[0, d] = sidi[0, d] + hoff
                gidx[1, d] = sidi[1, d]
            pltpu.async_copy(hcat.at[gidx.at[0]], hrows, semh)
            pltpu.async_copy(adtab.at[gidx.at[1]], adrows, sema)

        def consume(s):
            sidi, gidx, hrows, adrows, sbuf, semi, semh, sema = s
            pltpu.make_async_copy(hcat.at[gidx.at[0]], hrows, semh).wait()
            pltpu.make_async_copy(adtab.at[gidx.at[1]], adrows,
                                  sema).wait()
            for g in range(CH // D):
                ridx = iota + g * D
                a_s = jnp.zeros((D,), jnp.float32)
                hcols = []
                for c in range(D):
                    hc = plsc.load_gather(
                        hrows, [ridx, jnp.full((D,), c, jnp.int32)])
                    hcols.append(hc)
                    a_s = a_s + hc * asrc_s[c]
                a_d = plsc.load_gather(adrows, [ridx, ccol])
                al = a_s + a_d
                al = jnp.where(al >= 0, al, al * jnp.float32(0.2))
                w = jnp.exp(al)
                plsc.store_scatter(
                    sbuf, [ridx, jnp.full((D,), D, jnp.int32)], w)
                for c in range(D):
                    plsc.store_scatter(
                        sbuf, [ridx, jnp.full((D,), c, jnp.int32)],
                        hcols[c] * w)
            pltpu.sync_copy(sbuf, acc.at[gidx.at[1]], add=True)

        fire_idx(0, sets[0])
        fire_idx(1, sets[1])
        prep_fire(sets[0])
        fire_idx(2, sets[0])

        def pair_body(i2, _):
            c0 = 2 * i2
            prep_fire(sets[1])

            @pl.when(c0 + 3 < cpt)
            def _():
                fire_idx(c0 + 3, sets[1])

            consume(sets[0])

            @pl.when(c0 + 2 < cpt)
            def _():
                prep_fire(sets[0])

            @pl.when(c0 + 4 < cpt)
            def _():
                fire_idx(c0 + 4, sets[0])

            consume(sets[1])
            return 0

        lax.fori_loop(0, cpt // 2, pair_body, 0)
        plsc.subcore_barrier()

        def wb_body(z, _):
            r0 = sid * ROWPT + z * ZR
            pltpu.sync_copy(acc.at[pl.ds(r0, ZR)],
                            out.at[conv, pl.ds(r0, ZR)])
            return 0
        lax.fori_loop(0, ROWPT // ZR, wb_body, 0)
        plsc.subcore_barrier()


def kernel(x, edge_index, W_F1, asrc_F1, adst_F1, b_F1,
           W_F2, asrc_F2, adst_F2, b_F2,
           W_G1, asrc_G1, adst_G1, b_G1,
           W_G2, asrc_G2, adst_G2, b_G2):
    n = x.shape[0]
    e = edge_index.shape[1]
    npad = ((n + NTILE * ZR - 1) // (NTILE * ZR)) * (NTILE * ZR)
    if npad == n:
        npad += NTILE * ZR
    et = e + n
    chunks_per_tile = -(-et // (CH * NTILE))
    et_pad = chunks_per_tile * CH * NTILE

    idt = edge_index.dtype
    loops = jnp.arange(n, dtype=idt)
    fill = jnp.full((et_pad - et,), n, dtype=idt)
    srcs = jnp.concatenate([edge_index[0], loops, fill]).astype(jnp.int32)
    dsts = jnp.concatenate([edge_index[1], loops, fill]).astype(jnp.int32)
    ct = chunks_per_tile * NTILE
    edges3 = jnp.stack([srcs.reshape(ct, CH), dsts.reshape(ct, CH)], axis=1)

    x_pad = jnp.zeros((npad, 2 * D), jnp.float32).at[:n].set(x)
    wcat = jnp.stack([W_F1, W_F2, W_G1, W_G2])
    adcat = jnp.stack([adst_F1, adst_F2, adst_G1, adst_G2])
    asrct = jnp.stack([asrc_F1, asrc_F2, asrc_G1, asrc_G2])
    btab = jnp.stack([b_F1, b_F2, b_G1, b_G2])
    zeros_h = jnp.zeros((ZR, 2 * D), jnp.float32)

    blk = 256
    grid = (npad // blk,)
    h_sh = jax.ShapeDtypeStruct((npad, D), jnp.float32)
    h0, h1, h2, h3, adtab = pl.pallas_call(
        _prep_body,
        grid=grid,
        in_specs=[
            pl.BlockSpec((blk, 2 * D), lambda i: (i, 0)),
            pl.BlockSpec((4, D, D), lambda i: (0, 0, 0)),
            pl.BlockSpec((4, D), lambda i: (0, 0)),
        ],
        out_specs=[pl.BlockSpec((blk, D), lambda i: (i, 0))] * 5,
        out_shape=[h_sh, h_sh, h_sh, h_sh, h_sh],
    )(x_pad, wcat, adcat)
    hcat = jnp.concatenate([h0, h1, h2, h3], axis=0)

    mesh = plsc.VectorSubcoreMesh(core_axis_name="c", subcore_axis_name="s")
    sc = functools.partial(
        pl.kernel,
        out_type=jax.ShapeDtypeStruct((4, npad, 2 * D), jnp.float32),
        mesh=mesh,
        scratch_types=[
            pltpu.VMEM_SHARED((npad, 2 * D), jnp.float32),
            pltpu.VMEM((2, CH), jnp.int32),
            pltpu.VMEM((2, CH), jnp.int32),
            pltpu.VMEM((2, CH), jnp.int32),
            pltpu.VMEM((2, CH), jnp.int32),
            pltpu.VMEM((CH, D), jnp.float32),
            pltpu.VMEM((CH, D), jnp.float32),
            pltpu.VMEM((CH, D), jnp.float32),
            pltpu.VMEM((CH, D), jnp.float32),
            pltpu.VMEM((CH, 2 * D), jnp.float32),
            pltpu.VMEM((CH, 2 * D), jnp.float32),
            pltpu.VMEM((D,), jnp.float32),
            pltpu.SemaphoreType.DMA,
            pltpu.SemaphoreType.DMA,
            pltpu.SemaphoreType.DMA,
            pltpu.SemaphoreType.DMA,
            pltpu.SemaphoreType.DMA,
            pltpu.SemaphoreType.DMA,
        ],
        compiler_params=pltpu.CompilerParams(
            needs_layout_passes=False, use_tc_tiling_on_sc=False),
    )(functools.partial(_sc_body, chunks_per_tile=chunks_per_tile,
                        npad=npad))
    accs = sc(hcat, adtab, asrct, edges3, zeros_h)

    x1n, x2n, ld = pl.pallas_call(
        _fin_body,
        grid=grid,
        in_specs=[
            pl.BlockSpec((4, blk, 2 * D), lambda i: (0, i, 0)),
            pl.BlockSpec((blk, 2 * D), lambda i: (i, 0)),
            pl.BlockSpec((4, D), lambda i: (0, 0)),
        ],
        out_specs=[
            pl.BlockSpec((blk, D), lambda i: (i, 0)),
            pl.BlockSpec((blk, D), lambda i: (i, 0)),
            pl.BlockSpec((blk, 1), lambda i: (i, 0)),
        ],
        out_shape=[
            jax.ShapeDtypeStruct((npad, D), jnp.float32),
            jax.ShapeDtypeStruct((npad, D), jnp.float32),
            jax.ShapeDtypeStruct((npad, 1), jnp.float32),
        ],
    )(accs, x_pad, btab)

    return x1n[:n], x2n[:n], ld[:n, 0]


# speedup vs baseline: 82.3351x; 1.5668x over previous
import functools

import jax
import jax.numpy as jnp
from jax import lax
from jax.experimental import pallas as pl
from jax.experimental.pallas import tpu as pltpu
from jax.experimental.pallas import tpu_sc as plsc

D = 16
NSC = 2
NTILE = 16
CH = 128
ROWPT = 3200
ZR = 400


def _prep_body(x_ref, wcat_ref, adcat_ref, h0, h1, h2, h3, ad_ref):
    xb = x_ref[...]
    x1 = xb[:, :D]
    x2 = xb[:, D:]
    outs = (h0, h1, h2, h3)
    cols = []
    for c in range(4):
        xh = x1 if c < 2 else x2
        h = jnp.dot(xh, wcat_ref[c], preferred_element_type=jnp.float32)
        outs[c][...] = h
        cols.append((h * adcat_ref[c][None, :]).sum(-1, keepdims=True))
    zero = jnp.zeros((xb.shape[0], D - 4), jnp.float32)
    ad_ref[...] = jnp.concatenate(cols + [zero], axis=1)


def _fin_body(num_ref, den_ref, x_ref, btab_ref, x1n_ref, x2n_ref, ld_ref):
    def conv_out(c):
        return num_ref[c] / den_ref[c] + btab_ref[c][None, :]

    s1 = conv_out(0)
    t1 = conv_out(1)
    s2 = conv_out(2)
    t2 = conv_out(3)
    x2 = x_ref[:, D:]
    x1n = x2 * jnp.exp(s1) + t1
    x2n = x1n * jnp.exp(s2) + t2
    x1n_ref[...] = x1n
    x2n_ref[...] = x2n
    ld_ref[...] = (s1 + s2).sum(axis=1, keepdims=True)


def _sc_body(hcat, adcolt, asrct, edges3, zeros_h, outn, outd,
             acc, den, sidi0, sidi1, gidx0, gidx1, hrows0, hrows1,
             sbuf0, sbuf1, sden0, sden1, adloc, asrcv,
             semi0, semi1, semh0, semh1,
             chunks_per_tile, npad):
    cid = lax.axis_index("c")
    sid = lax.axis_index("s")
    iota = lax.iota(jnp.int32, D)
    cpt = chunks_per_tile
    zero16 = jnp.zeros((D,), jnp.float32)
    sets = ((sidi0, gidx0, hrows0, sbuf0, sden0, semi0, semh0),
            (sidi1, gidx1, hrows1, sbuf1, sden1, semi1, semh1))

    def fire_idx(j, s):
        sidi = s[0]
        semi = s[5]
        pltpu.async_copy(edges3.at[sid * cpt + j], sidi, semi)

    for p in range(2):
        conv = 2 * cid + p

        def zero_body(z, _):
            r0 = sid * ROWPT + z * ZR
            pltpu.sync_copy(zeros_h, acc.at[pl.ds(r0, ZR)])
            return 0
        lax.fori_loop(0, ROWPT // ZR, zero_body, 0)
        pltpu.sync_copy(zeros_h.at[pl.ds(0, ROWPT // D)],
                        den.at[pl.ds(sid * (ROWPT // D), ROWPT // D)])
        plsc.subcore_barrier()

        pltpu.sync_copy(asrct.at[conv], asrcv)
        pltpu.sync_copy(adcolt.at[conv], adloc)
        av = asrcv[...]
        asrc_s = [av[c] for c in range(D)]
        hoff = jnp.broadcast_to(conv * npad, (D,)).astype(jnp.int32)

        def prep_fire(s):
            sidi, gidx, hrows = s[0], s[1], s[2]
            semi, semh = s[5], s[6]
            pltpu.make_async_copy(edges3.at[0], sidi, semi).wait()
            for g in range(CH // D):
                d = pl.ds(g * D, D)
                gidx[0, d] = sidi[0, d] + hoff
                gidx[1, d] = sidi[1, d]
                gidx[2, d] = lax.shift_right_logical(sidi[1, d], 4)
            pltpu.async_copy(hcat.at[gidx.at[0]], hrows, semh)

        def consume(s):
            gidx, hrows, sbuf, sden = s[1], s[2], s[3], s[4]
            semh = s[6]
            pltpu.make_async_copy(hcat.at[gidx.at[0]], hrows, semh).wait()
            for g in range(CH // D):
                ridx = iota + g * D
                a_s = jnp.zeros((D,), jnp.float32)
                hcols = []
                for c in range(D):
                    hc = plsc.load_gather(
                        hrows, [ridx, jnp.full((D,), c, jnp.int32)])
                    hcols.append(hc)
                    a_s = a_s + hc * asrc_s[c]
                dstv = gidx[1, pl.ds(g * D, D)]
                a_d = plsc.load_gather(adloc, [dstv])
                al = a_s + a_d
                al = jnp.where(al >= 0, al, al * jnp.float32(0.2))
                w = jnp.exp(al)
                for c in range(D):
                    plsc.store_scatter(
                        sbuf, [ridx, jnp.full((D,), c, jnp.int32)],
                        hcols[c] * w)
                for r in range(D):
                    sden[g * D + r, :] = zero16
                plsc.store_scatter(
                    sden, [ridx, dstv & jnp.int32(D - 1)], w)
            pltpu.sync_copy(sbuf, acc.at[gidx.at[1]], add=True)
            pltpu.sync_copy(sden, den.at[gidx.at[2]], add=True)

        fire_idx(0, sets[0])
        fire_idx(1, sets[1])
        prep_fire(sets[0])
        fire_idx(2, sets[0])

        def pair_body(i2, _):
            c0 = 2 * i2
            prep_fire(sets[1])

            @pl.when(c0 + 3 < cpt)
            def _():
                fire_idx(c0 + 3, sets[1])

            consume(sets[0])

            @pl.when(c0 + 2 < cpt)
            def _():
                prep_fire(sets[0])

            @pl.when(c0 + 4 < cpt)
            def _():
                fire_idx(c0 + 4, sets[0])

            consume(sets[1])
            return 0

        lax.fori_loop(0, cpt // 2, pair_body, 0)
        plsc.subcore_barrier()

        def wb_body(z, _):
            r0 = sid * ROWPT + z * ZR
            pltpu.sync_copy(acc.at[pl.ds(r0, ZR)],
                            outn.at[conv, pl.ds(r0, ZR)])
            return 0
        lax.fori_loop(0, ROWPT // ZR, wb_body, 0)
        d0 = sid * (ROWPT // D)
        pltpu.sync_copy(den.at[pl.ds(d0, ROWPT // D)],
                        outd.at[conv, pl.ds(d0, ROWPT // D)])
        plsc.subcore_barrier()


def kernel(x, edge_index, W_F1, asrc_F1, adst_F1, b_F1,
           W_F2, asrc_F2, adst_F2, b_F2,
           W_G1, asrc_G1, adst_G1, b_G1,
           W_G2, asrc_G2, adst_G2, b_G2):
    n = x.shape[0]
    e = edge_index.shape[1]
    npad = ((n + NTILE * ZR - 1) // (NTILE * ZR)) * (NTILE * ZR)
    if npad == n:
        npad += NTILE * ZR
    et = e + n
    chunks_per_tile = -(-et // (CH * NTILE))
    et_pad = chunks_per_tile * CH * NTILE

    idt = edge_index.dtype
    loops = jnp.arange(n, dtype=idt)
    fill = jnp.full((et_pad - et,), n, dtype=idt)
    srcs = jnp.concatenate([edge_index[0], loops, fill]).astype(jnp.int32)
    dsts = jnp.concatenate([edge_index[1], loops, fill]).astype(jnp.int32)
    ct = chunks_per_tile * NTILE
    edges3 = jnp.stack([srcs.reshape(ct, CH), dsts.reshape(ct, CH)], axis=1)

    x_pad = jnp.zeros((npad, 2 * D), jnp.float32).at[:n].set(x)
    wcat = jnp.stack([W_F1, W_F2, W_G1, W_G2])
    adcat = jnp.stack([adst_F1, adst_F2, adst_G1, adst_G2])
    asrct = jnp.stack([asrc_F1, asrc_F2, asrc_G1, asrc_G2])
    btab = jnp.stack([b_F1, b_F2, b_G1, b_G2])
    zeros_h = jnp.zeros((ZR, D), jnp.float32)

    blk = 256
    grid = (npad // blk,)
    h_sh = jax.ShapeDtypeStruct((npad, D), jnp.float32)
    h0, h1, h2, h3, adtab = pl.pallas_call(
        _prep_body,
        grid=grid,
        in_specs=[
            pl.BlockSpec((blk, 2 * D), lambda i: (i, 0)),
            pl.BlockSpec((4, D, D), lambda i: (0, 0, 0)),
            pl.BlockSpec((4, D), lambda i: (0, 0)),
        ],
        out_specs=[pl.BlockSpec((blk, D), lambda i: (i, 0))] * 5,
        out_shape=[h_sh, h_sh, h_sh, h_sh, h_sh],
    )(x_pad, wcat, adcat)
    hcat = jnp.concatenate([h0, h1, h2, h3], axis=0)

    mesh = plsc.VectorSubcoreMesh(core_axis_name="c", subcore_axis_name="s")
    sc = functools.partial(
        pl.kernel,
        out_type=(jax.ShapeDtypeStruct((4, npad, D), jnp.float32),
                  jax.ShapeDtypeStruct((4, npad // D, D), jnp.float32)),
        mesh=mesh,
        scratch_types=[
            pltpu.VMEM_SHARED((npad, D), jnp.float32),
            pltpu.VMEM_SHARED((npad // D, D), jnp.float32),
            pltpu.VMEM((2, CH), jnp.int32),
            pltpu.VMEM((2, CH), jnp.int32),
            pltpu.VMEM((3, CH), jnp.int32),
            pltpu.VMEM((3, CH), jnp.int32),
            pltpu.VMEM((CH, D), jnp.float32),
            pltpu.VMEM((CH, D), jnp.float32),
            pltpu.VMEM((CH, D), jnp.float32),
            pltpu.VMEM((CH, D), jnp.float32),
            pltpu.VMEM((CH, D), jnp.float32),
            pltpu.VMEM((CH, D), jnp.float32),
            pltpu.VMEM((npad,), jnp.float32),
            pltpu.VMEM((D,), jnp.float32),
            pltpu.SemaphoreType.DMA,
            pltpu.SemaphoreType.DMA,
            pltpu.SemaphoreType.DMA,
            pltpu.SemaphoreType.DMA,
        ],
        compiler_params=pltpu.CompilerParams(
            needs_layout_passes=False, use_tc_tiling_on_sc=False),
    )(functools.partial(_sc_body, chunks_per_tile=chunks_per_tile,
                        npad=npad))
    adcolt = jnp.transpose(adtab[:, :4])
    nums, dens = sc(hcat, adcolt, asrct, edges3, zeros_h)
    densr = dens.reshape(4, npad, 1)

    x1n, x2n, ld = pl.pallas_call(
        _fin_body,
        grid=grid,
        in_specs=[
            pl.BlockSpec((4, blk, D), lambda i: (0, i, 0)),
            pl.BlockSpec((4, blk, 1), lambda i: (0, i, 0)),
            pl.BlockSpec((blk, 2 * D), lambda i: (i, 0)),
            pl.BlockSpec((4, D), lambda i: (0, 0)),
        ],
        out_specs=[
            pl.BlockSpec((blk, D), lambda i: (i, 0)),
            pl.BlockSpec((blk, D), lambda i: (i, 0)),
            pl.BlockSpec((blk, 1), lambda i: (i, 0)),
        ],
        out_shape=[
            jax.ShapeDtypeStruct((npad, D), jnp.float32),
            jax.ShapeDtypeStruct((npad, D), jnp.float32),
            jax.ShapeDtypeStruct((npad, 1), jnp.float32),
        ],
    )(nums, densr, x_pad, btab)

    return x1n[:n], x2n[:n], ld[:n, 0]


# speedup vs baseline: 88.3489x; 1.0730x over previous
import functools

import jax
import jax.numpy as jnp
from jax import lax
from jax.experimental import pallas as pl
from jax.experimental.pallas import tpu as pltpu
from jax.experimental.pallas import tpu_sc as plsc

D = 16
NSC = 2
NTILE = 16
CH = 128
ROWPT = 3200
ZR = 400


def _prep_body(x_ref, wcat_ref, adcat_ref, h0, h1, h2, h3, ad_ref):
    xb = x_ref[...]
    x1 = xb[:, :D]
    x2 = xb[:, D:]
    outs = (h0, h1, h2, h3)
    cols = []
    for c in range(4):
        xh = x1 if c < 2 else x2
        h = jnp.dot(xh, wcat_ref[c], preferred_element_type=jnp.float32)
        outs[c][...] = h
        cols.append((h * adcat_ref[c][None, :]).sum(-1, keepdims=True))
    zero = jnp.zeros((xb.shape[0], D - 4), jnp.float32)
    ad_ref[...] = jnp.concatenate(cols + [zero], axis=1)


def _fin_body(num_ref, den_ref, x_ref, btab_ref, x1n_ref, x2n_ref, ld_ref):
    def conv_out(c):
        return num_ref[c] / den_ref[c] + btab_ref[c][None, :]

    s1 = conv_out(0)
    t1 = conv_out(1)
    s2 = conv_out(2)
    t2 = conv_out(3)
    x2 = x_ref[:, D:]
    x1n = x2 * jnp.exp(s1) + t1
    x2n = x1n * jnp.exp(s2) + t2
    x1n_ref[...] = x1n
    x2n_ref[...] = x2n
    ld_ref[...] = (s1 + s2).sum(axis=1, keepdims=True)


def _sc_body(hcat, adcolt, asrct, edges3, zeros_h, outn, outd,
             acc, den, sidi0, sidi1, gidx0, gidx1, hrows0, hrows1,
             sbuf0, sbuf1, sden0, sden1, adloc, asrcv,
             semi0, semi1, semh0, semh1,
             chunks_per_tile, npad):
    cid = lax.axis_index("c")
    sid = lax.axis_index("s")
    iota = lax.iota(jnp.int32, D)
    cpt = chunks_per_tile
    zero16 = jnp.zeros((D,), jnp.float32)
    sets = ((sidi0, gidx0, hrows0, sbuf0, sden0, semi0, semh0),
            (sidi1, gidx1, hrows1, sbuf1, sden1, semi1, semh1))

    def fire_idx(j, s):
        sidi = s[0]
        semi = s[5]
        pltpu.async_copy(edges3.at[sid * cpt + j], sidi, semi)

    for p in range(2):
        conv = 2 * cid + p

        def zero_body(z, _):
            r0 = sid * ROWPT + z * ZR
            pltpu.sync_copy(zeros_h, acc.at[pl.ds(r0, ZR)])
            return 0
        lax.fori_loop(0, ROWPT // ZR, zero_body, 0)
        pltpu.sync_copy(zeros_h.at[pl.ds(0, ROWPT // D)],
                        den.at[pl.ds(sid * (ROWPT // D), ROWPT // D)])
        plsc.subcore_barrier()

        pltpu.sync_copy(asrct.at[conv], asrcv)
        pltpu.sync_copy(adcolt.at[conv], adloc)
        av = asrcv[...]
        asrc_s = [av[c] for c in range(D)]
        hoff = jnp.broadcast_to(conv * npad, (D,)).astype(jnp.int32)

        def prep_fire(s):
            sidi, gidx, hrows = s[0], s[1], s[2]
            semi, semh = s[5], s[6]
            pltpu.make_async_copy(edges3.at[0], sidi, semi).wait()
            for g in range(CH // D):
                d = pl.ds(g * D, D)
                gidx[0, d] = sidi[0, d] + hoff
                gidx[1, d] = sidi[1, d]
                gidx[2, d] = lax.shift_right_logical(sidi[1, d], 4)
            pltpu.async_copy(hcat.at[gidx.at[0]], hrows, semh)

        def consume(s):
            gidx, hrows, sbuf, sden = s[1], s[2], s[3], s[4]
            semh = s[6]
            pltpu.make_async_copy(hcat.at[gidx.at[0]], hrows, semh).wait()
            for g in range(CH // D):
                ridx = iota + g * D
                a_s = jnp.zeros((D,), jnp.float32)
                hcols = []
                for c in range(D):
                    hc = plsc.load_gather(
                        hrows, [ridx, jnp.full((D,), c, jnp.int32)])
                    hcols.append(hc)
                    a_s = a_s + hc * asrc_s[c]
                dstv = gidx[1, pl.ds(g * D, D)]
                a_d = plsc.load_gather(adloc, [dstv])
                al = a_s + a_d
                al = jnp.where(al >= 0, al, al * jnp.float32(0.2))
                w = jnp.exp(al)
                for c in range(D):
                    plsc.store_scatter(
                        sbuf, [ridx, jnp.full((D,), c, jnp.int32)],
                        hcols[c] * w)
                for r in range(D):
                    sden[g * D + r, :] = zero16
                plsc.store_scatter(
                    sden, [ridx, dstv & jnp.int32(D - 1)], w)
            pltpu.sync_copy(sbuf, acc.at[gidx.at[1]], add=True)
            pltpu.sync_copy(sden, den.at[gidx.at[2]], add=True)

        fire_idx(0, sets[0])
        fire_idx(1, sets[1])
        prep_fire(sets[0])
        fire_idx(2, sets[0])

        def pair_body(i2, _):
            c0 = 2 * i2
            prep_fire(sets[1])

            @pl.when(c0 + 3 < cpt)
            def _():
                fire_idx(c0 + 3, sets[1])

            consume(sets[0])

            @pl.when(c0 + 2 < cpt)
            def _():
                prep_fire(sets[0])

            @pl.when(c0 + 4 < cpt)
            def _():
                fire_idx(c0 + 4, sets[0])

            consume(sets[1])
            return 0

        lax.fori_loop(0, cpt // 2, pair_body, 0)
        plsc.subcore_barrier()

        def wb_body(z, _):
            r0 = sid * ROWPT + z * ZR
            pltpu.sync_copy(acc.at[pl.ds(r0, ZR)],
                            outn.at[conv, pl.ds(r0, ZR)])
            return 0
        lax.fori_loop(0, ROWPT // ZR, wb_body, 0)
        d0 = sid * (ROWPT // D)
        pltpu.sync_copy(den.at[pl.ds(d0, ROWPT // D)],
                        outd.at[conv, pl.ds(d0, ROWPT // D)])
        plsc.subcore_barrier()


def kernel(x, edge_index, W_F1, asrc_F1, adst_F1, b_F1,
           W_F2, asrc_F2, adst_F2, b_F2,
           W_G1, asrc_G1, adst_G1, b_G1,
           W_G2, asrc_G2, adst_G2, b_G2):
    n = x.shape[0]
    e = edge_index.shape[1]
    npad = ((n + NTILE * ZR - 1) // (NTILE * ZR)) * (NTILE * ZR)
    if npad == n:
        npad += NTILE * ZR
    et = e + n
    chunks_per_tile = -(-et // (CH * NTILE))
    et_pad = chunks_per_tile * CH * NTILE

    idt = edge_index.dtype
    loops = jnp.arange(n, dtype=idt)
    fill = jnp.full((et_pad - et,), n, dtype=idt)
    srcs = jnp.concatenate([edge_index[0], loops, fill]).astype(jnp.int32)
    dsts = jnp.concatenate([edge_index[1], loops, fill]).astype(jnp.int32)
    ct = chunks_per_tile * NTILE
    edges3 = jnp.stack([srcs.reshape(ct, CH), dsts.reshape(ct, CH)], axis=1)

    x_pad = jnp.zeros((npad, 2 * D), jnp.float32).at[:n].set(x)
    wcat = jnp.stack([W_F1, W_F2, W_G1, W_G2])
    adcat = jnp.stack([adst_F1, adst_F2, adst_G1, adst_G2])
    asrct = jnp.stack([asrc_F1, asrc_F2, asrc_G1, asrc_G2])
    btab = jnp.stack([b_F1, b_F2, b_G1, b_G2])
    zeros_h = jnp.zeros((ZR, D), jnp.float32)

    blk = 1600
    grid = (npad // blk,)
    h_sh = jax.ShapeDtypeStruct((npad, D), jnp.float32)
    h0, h1, h2, h3, adtab = pl.pallas_call(
        _prep_body,
        grid=grid,
        in_specs=[
            pl.BlockSpec((blk, 2 * D), lambda i: (i, 0)),
            pl.BlockSpec((4, D, D), lambda i: (0, 0, 0)),
            pl.BlockSpec((4, D), lambda i: (0, 0)),
        ],
        out_specs=[pl.BlockSpec((blk, D), lambda i: (i, 0))] * 5,
        out_shape=[h_sh, h_sh, h_sh, h_sh, h_sh],
    )(x_pad, wcat, adcat)
    hcat = jnp.concatenate([h0, h1, h2, h3], axis=0)

    mesh = plsc.VectorSubcoreMesh(core_axis_name="c", subcore_axis_name="s")
    sc = functools.partial(
        pl.kernel,
        out_type=(jax.ShapeDtypeStruct((4, npad, D), jnp.float32),
                  jax.ShapeDtypeStruct((4, npad // D, D), jnp.float32)),
        mesh=mesh,
        scratch_types=[
            pltpu.VMEM_SHARED((npad, D), jnp.float32),
            pltpu.VMEM_SHARED((npad // D, D), jnp.float32),
            pltpu.VMEM((2, CH), jnp.int32),
            pltpu.VMEM((2, CH), jnp.int32),
            pltpu.VMEM((3, CH), jnp.int32),
            pltpu.VMEM((3, CH), jnp.int32),
            pltpu.VMEM((CH, D), jnp.float32),
            pltpu.VMEM((CH, D), jnp.float32),
            pltpu.VMEM((CH, D), jnp.float32),
            pltpu.VMEM((CH, D), jnp.float32),
            pltpu.VMEM((CH, D), jnp.float32),
            pltpu.VMEM((CH, D), jnp.float32),
            pltpu.VMEM((npad,), jnp.float32),
            pltpu.VMEM((D,), jnp.float32),
            pltpu.SemaphoreType.DMA,
            pltpu.SemaphoreType.DMA,
            pltpu.SemaphoreType.DMA,
            pltpu.SemaphoreType.DMA,
        ],
        compiler_params=pltpu.CompilerParams(
            needs_layout_passes=False, use_tc_tiling_on_sc=False),
    )(functools.partial(_sc_body, chunks_per_tile=chunks_per_tile,
                        npad=npad))
    adcolt = jnp.transpose(adtab[:, :4])
    nums, dens = sc(hcat, adcolt, asrct, edges3, zeros_h)
    densr = dens.reshape(4, npad, 1)

    x1n, x2n, ld = pl.pallas_call(
        _fin_body,
        grid=grid,
        in_specs=[
            pl.BlockSpec((4, blk, D), lambda i: (0, i, 0)),
            pl.BlockSpec((4, blk, 1), lambda i: (0, i, 0)),
            pl.BlockSpec((blk, 2 * D), lambda i: (i, 0)),
            pl.BlockSpec((4, D), lambda i: (0, 0)),
        ],
        out_specs=[
            pl.BlockSpec((blk, D), lambda i: (i, 0)),
            pl.BlockSpec((blk, D), lambda i: (i, 0)),
            pl.BlockSpec((blk, 1), lambda i: (i, 0)),
        ],
        out_shape=[
            jax.ShapeDtypeStruct((npad, D), jnp.float32),
            jax.ShapeDtypeStruct((npad, D), jnp.float32),
            jax.ShapeDtypeStruct((npad, 1), jnp.float32),
        ],
    )(nums, densr, x_pad, btab)

    return x1n[:n], x2n[:n], ld[:n, 0]


# speedup vs baseline: 95.5152x; 1.0811x over previous
import functools

import jax
import jax.numpy as jnp
from jax import lax
from jax.experimental import pallas as pl
from jax.experimental.pallas import tpu as pltpu
from jax.experimental.pallas import tpu_sc as plsc

D = 16
NSC = 2
NTILE = 16
CH = 128
ROWPT = 3200
ZR = 400


def _prep_body(x_ref, wcat_ref, adcat_ref, h4_ref, ad_ref):
    xb = x_ref[...]
    x1 = xb[:, :D]
    x2 = xb[:, D:]
    cols = []
    for c in range(4):
        xh = x1 if c < 2 else x2
        h = jnp.dot(xh, wcat_ref[c], preferred_element_type=jnp.float32)
        h4_ref[c] = h
        cols.append((h * adcat_ref[c][None, :]).sum(-1, keepdims=True))
    zero = jnp.zeros((xb.shape[0], D - 4), jnp.float32)
    ad_ref[...] = jnp.concatenate(cols + [zero], axis=1)


def _fin_body(num_ref, den_ref, x_ref, btab_ref, x1n_ref, x2n_ref, ld_ref):
    def conv_out(c):
        return num_ref[c] / den_ref[c] + btab_ref[c][None, :]

    s1 = conv_out(0)
    t1 = conv_out(1)
    s2 = conv_out(2)
    t2 = conv_out(3)
    x2 = x_ref[:, D:]
    x1n = x2 * jnp.exp(s1) + t1
    x2n = x1n * jnp.exp(s2) + t2
    x1n_ref[...] = x1n
    x2n_ref[...] = x2n
    ld_ref[...] = (s1 + s2).sum(axis=1, keepdims=True)


def _sc_body(hcat, adcolt, asrct, ei, zeros_h, outn, outd,
             acc, den, sidi0, sidi1, gidx0, gidx1, hrows0, hrows1,
             sbuf0, sbuf1, sden0, sden1, adloc, asrcv,
             semi0, semi1, semh0, semh1,
             chunks_per_tile, npad, ne, n):
    cid = lax.axis_index("c")
    sid = lax.axis_index("s")
    iota = lax.iota(jnp.int32, D)
    cpt = chunks_per_tile
    zero16 = jnp.zeros((D,), jnp.float32)
    sets = ((sidi0, gidx0, hrows0, sbuf0, sden0, semi0, semh0),
            (sidi1, gidx1, hrows1, sbuf1, sden1, semi1, semh1))

    def fire_idx(j, s):
        sidi = s[0]
        semi = s[5]
        base = (sid * cpt + j) * CH

        @pl.when(base < ne)
        def _():
            pltpu.async_copy(ei.at[pl.ds(0, 2), pl.ds(base, CH)],
                             sidi, semi)

    for p in range(2):
        conv = 2 * cid + p

        def zero_body(z, _):
            r0 = sid * ROWPT + z * ZR
            pltpu.sync_copy(zeros_h, acc.at[pl.ds(r0, ZR)])
            return 0
        lax.fori_loop(0, ROWPT // ZR, zero_body, 0)
        pltpu.sync_copy(zeros_h.at[pl.ds(0, ROWPT // D)],
                        den.at[pl.ds(sid * (ROWPT // D), ROWPT // D)])
        plsc.subcore_barrier()

        pltpu.sync_copy(asrct.at[conv], asrcv)
        pltpu.sync_copy(adcolt.at[conv], adloc)
        av = asrcv[...]
        asrc_s = [av[c] for c in range(D)]
        hoff = jnp.broadcast_to(conv * npad, (D,)).astype(jnp.int32)

        def prep_fire(jj, s):
            sidi, gidx, hrows = s[0], s[1], s[2]
            semi, semh = s[5], s[6]
            base = (sid * cpt + jj) * CH

            @pl.when(base < ne)
            def _():
                pltpu.make_async_copy(ei.at[pl.ds(0, 2), pl.ds(0, CH)],
                                      sidi, semi).wait()

            @pl.when(base >= ne)
            def _():
                for g in range(CH // D):
                    d = pl.ds(g * D, D)
                    v = jnp.minimum(iota + (base - ne + g * D),
                                    jnp.int32(n))
                    sidi[0, d] = v
                    sidi[1, d] = v

            for g in range(CH // D):
                d = pl.ds(g * D, D)
                gidx[0, d] = sidi[0, d] + hoff
                gidx[1, d] = sidi[1, d]
                gidx[2, d] = lax.shift_right_logical(sidi[1, d], 4)
            pltpu.async_copy(hcat.at[gidx.at[0]], hrows, semh)

        def consume(s):
            gidx, hrows, sbuf, sden = s[1], s[2], s[3], s[4]
            semh = s[6]
            pltpu.make_async_copy(hcat.at[gidx.at[0]], hrows, semh).wait()
            for g in range(CH // D):
                ridx = iota + g * D
                a_s = jnp.zeros((D,), jnp.float32)
                hcols = []
                for c in range(D):
                    hc = plsc.load_gather(
                        hrows, [ridx, jnp.full((D,), c, jnp.int32)])
                    hcols.append(hc)
                    a_s = a_s + hc * asrc_s[c]
                dstv = gidx[1, pl.ds(g * D, D)]
                a_d = plsc.load_gather(adloc, [dstv])
                al = a_s + a_d
                al = jnp.where(al >= 0, al, al * jnp.float32(0.2))
                w = jnp.exp(al)
                for c in range(D):
                    plsc.store_scatter(
                        sbuf, [ridx, jnp.full((D,), c, jnp.int32)],
                        hcols[c] * w)
                for r in range(D):
                    sden[g * D + r, :] = zero16
                plsc.store_scatter(
                    sden, [ridx, dstv & jnp.int32(D - 1)], w)
            pltpu.sync_copy(sbuf, acc.at[gidx.at[1]], add=True)
            pltpu.sync_copy(sden, den.at[gidx.at[2]], add=True)

        fire_idx(0, sets[0])
        fire_idx(1, sets[1])
        prep_fire(0, sets[0])
        fire_idx(2, sets[0])

        def pair_body(i2, _):
            c0 = 2 * i2
            prep_fire(c0 + 1, sets[1])

            @pl.when(c0 + 3 < cpt)
            def _():
                fire_idx(c0 + 3, sets[1])

            consume(sets[0])

            @pl.when(c0 + 2 < cpt)
            def _():
                prep_fire(c0 + 2, sets[0])

            @pl.when(c0 + 4 < cpt)
            def _():
                fire_idx(c0 + 4, sets[0])

            consume(sets[1])
            return 0

        lax.fori_loop(0, cpt // 2, pair_body, 0)
        plsc.subcore_barrier()

        def wb_body(z, _):
            r0 = sid * ROWPT + z * ZR
            pltpu.sync_copy(acc.at[pl.ds(r0, ZR)],
                            outn.at[conv, pl.ds(r0, ZR)])
            return 0
        lax.fori_loop(0, ROWPT // ZR, wb_body, 0)
        d0 = sid * (ROWPT // D)
        pltpu.sync_copy(den.at[pl.ds(d0, ROWPT // D)],
                        outd.at[conv, pl.ds(d0, ROWPT // D)])
        plsc.subcore_barrier()


def kernel(x, edge_index, W_F1, asrc_F1, adst_F1, b_F1,
           W_F2, asrc_F2, adst_F2, b_F2,
           W_G1, asrc_G1, adst_G1, b_G1,
           W_G2, asrc_G2, adst_G2, b_G2):
    n = x.shape[0]
    e = edge_index.shape[1]
    npad = ((n + NTILE * ZR - 1) // (NTILE * ZR)) * (NTILE * ZR)
    if npad == n:
        npad += NTILE * ZR
    et = e + n
    chunks_per_tile = -(-et // (CH * NTILE))
    et_pad = chunks_per_tile * CH * NTILE

    del et_pad
    ei = edge_index.astype(jnp.int32)

    x_pad = jnp.zeros((npad, 2 * D), jnp.float32).at[:n].set(x)
    wcat = jnp.stack([W_F1, W_F2, W_G1, W_G2])
    adcat = jnp.stack([adst_F1, adst_F2, adst_G1, adst_G2])
    asrct = jnp.stack([asrc_F1, asrc_F2, asrc_G1, asrc_G2])
    btab = jnp.stack([b_F1, b_F2, b_G1, b_G2])
    zeros_h = jnp.zeros((ZR, D), jnp.float32)

    blk = 1600
    grid = (npad // blk,)
    h4, adtab = pl.pallas_call(
        _prep_body,
        grid=grid,
        in_specs=[
            pl.BlockSpec((blk, 2 * D), lambda i: (i, 0)),
            pl.BlockSpec((4, D, D), lambda i: (0, 0, 0)),
            pl.BlockSpec((4, D), lambda i: (0, 0)),
        ],
        out_specs=[
            pl.BlockSpec((4, blk, D), lambda i: (0, i, 0)),
            pl.BlockSpec((blk, D), lambda i: (i, 0)),
        ],
        out_shape=[
            jax.ShapeDtypeStruct((4, npad, D), jnp.float32),
            jax.ShapeDtypeStruct((npad, D), jnp.float32),
        ],
    )(x_pad, wcat, adcat)
    hcat = h4.reshape(4 * npad, D)

    mesh = plsc.VectorSubcoreMesh(core_axis_name="c", subcore_axis_name="s")
    sc = functools.partial(
        pl.kernel,
        out_type=(jax.ShapeDtypeStruct((4, npad, D), jnp.float32),
                  jax.ShapeDtypeStruct((4, npad // D, D), jnp.float32)),
        mesh=mesh,
        scratch_types=[
            pltpu.VMEM_SHARED((npad, D), jnp.float32),
            pltpu.VMEM_SHARED((npad // D, D), jnp.float32),
            pltpu.VMEM((2, CH), jnp.int32),
            pltpu.VMEM((2, CH), jnp.int32),
            pltpu.VMEM((3, CH), jnp.int32),
            pltpu.VMEM((3, CH), jnp.int32),
            pltpu.VMEM((CH, D), jnp.float32),
            pltpu.VMEM((CH, D), jnp.float32),
            pltpu.VMEM((CH, D), jnp.float32),
            pltpu.VMEM((CH, D), jnp.float32),
            pltpu.VMEM((CH, D), jnp.float32),
            pltpu.VMEM((CH, D), jnp.float32),
            pltpu.VMEM((npad,), jnp.float32),
            pltpu.VMEM((D,), jnp.float32),
            pltpu.SemaphoreType.DMA,
            pltpu.SemaphoreType.DMA,
            pltpu.SemaphoreType.DMA,
            pltpu.SemaphoreType.DMA,
        ],
        compiler_params=pltpu.CompilerParams(
            needs_layout_passes=False, use_tc_tiling_on_sc=False),
    )(functools.partial(_sc_body, chunks_per_tile=chunks_per_tile,
                        npad=npad, ne=e, n=n))
    adcolt = jnp.transpose(adtab[:, :4])
    nums, dens = sc(hcat, adcolt, asrct, ei, zeros_h)
    densr = dens.reshape(4, npad, 1)

    x1n, x2n, ld = pl.pallas_call(
        _fin_body,
        grid=grid,
        in_specs=[
            pl.BlockSpec((4, blk, D), lambda i: (0, i, 0)),
            pl.BlockSpec((4, blk, 1), lambda i: (0, i, 0)),
            pl.BlockSpec((blk, 2 * D), lambda i: (i, 0)),
            pl.BlockSpec((4, D), lambda i: (0, 0)),
        ],
        out_specs=[
            pl.BlockSpec((blk, D), lambda i: (i, 0)),
            pl.BlockSpec((blk, D), lambda i: (i, 0)),
            pl.BlockSpec((blk, 1), lambda i: (i, 0)),
        ],
        out_shape=[
            jax.ShapeDtypeStruct((npad, D), jnp.float32),
            jax.ShapeDtypeStruct((npad, D), jnp.float32),
            jax.ShapeDtypeStruct((npad, 1), jnp.float32),
        ],
    )(nums, densr, x_pad, btab)

    return x1n[:n], x2n[:n], ld[:n, 0]
